# baseline (speedup 1.0000x reference)
import dataclasses
import functools

import jax
import jax.numpy as jnp
from jax import lax
from jax.experimental import pallas as pl
from jax.experimental.pallas import tpu as pltpu
from jax.experimental.pallas import tpu_sc as plsc

N = 10000
NPAD = 10016
NLOC = 313
NW = 32
E_TOT = 330000
E_PAD = 335872
EW = E_PAD // NW
SB = 4096
NBLK = E_PAD // SB
CAP = SB + 80
GB = 64
HI = jax.lax.Precision.HIGHEST


def _mesh():
    return plsc.VectorSubcoreMesh(core_axis_name="c", subcore_axis_name="s")


def _sc_params():
    cp = pltpu.CompilerParams()
    if "needs_layout_passes" in pltpu.CompilerParams.__dataclass_fields__:
        cp = dataclasses.replace(cp, needs_layout_passes=False)
    return cp


def _edge_weights(al_src_p, al_dst_p, srcp, dstp):
    @functools.partial(
        pl.kernel,
        out_type=jax.ShapeDtypeStruct((E_PAD,), jnp.float32),
        mesh=_mesh(),
        compiler_params=_sc_params(),
        scratch_types=[
            pltpu.VMEM((NPAD,), jnp.float32),
            pltpu.VMEM((NPAD,), jnp.float32),
            pltpu.VMEM((EW,), jnp.int32),
            pltpu.VMEM((EW,), jnp.int32),
            pltpu.VMEM((EW,), jnp.float32),
        ],
    )
    def k(als_hbm, ald_hbm, src_hbm, dst_hbm, w_hbm, als_v, ald_v, s_v, d_v, w_v):
        wid = lax.axis_index("s") * 2 + lax.axis_index("c")
        base = wid * EW
        pltpu.sync_copy(als_hbm, als_v)
        pltpu.sync_copy(ald_hbm, ald_v)
        pltpu.sync_copy(src_hbm.at[pl.ds(base, EW)], s_v)
        pltpu.sync_copy(dst_hbm.at[pl.ds(base, EW)], d_v)

        @pl.loop(0, EW, step=16)
        def _(i):
            s = s_v[pl.ds(i, 16)]
            d = d_v[pl.ds(i, 16)]
            t = plsc.load_gather(als_v, [s]) + plsc.load_gather(ald_v, [d])
            t = jnp.where(t >= 0.0, t, 0.2 * t)
            w_v[pl.ds(i, 16)] = jnp.exp(t)

        pltpu.sync_copy(w_v, w_hbm.at[pl.ds(base, EW)])

    return k(al_src_p, al_dst_p, srcp, dstp)


def _scatter(h, srcp, dstp, w, D):
    ACCN = (NLOC + 1) * D
    ACCD = (NLOC + 1) * 16

    @functools.partial(
        pl.kernel,
        out_type=(
            jax.ShapeDtypeStruct((NPAD * D,), jnp.float32),
            jax.ShapeDtypeStruct((NPAD * 16,), jnp.float32),
        ),
        mesh=_mesh(),
        compiler_params=_sc_params(),
        scratch_types=[
            pltpu.VMEM((ACCN,), jnp.float32),
            pltpu.VMEM((ACCD,), jnp.float32),
            pltpu.VMEM((SB,), jnp.int32),
            pltpu.VMEM((SB,), jnp.int32),
            pltpu.VMEM((SB,), jnp.float32),
            pltpu.VMEM((CAP,), jnp.int32),
            pltpu.VMEM((CAP,), jnp.int32),
            pltpu.VMEM((CAP,), jnp.float32),
            pltpu.VMEM((GB, D), jnp.float32),
            pltpu.SemaphoreType.DMA,
        ],
    )
    def k(h_hbm, src_hbm, dst_hbm, w_hbm, num_hbm, den_hbm,
          accn, accd, st_s, st_d, st_w, pd_s, pd_d, pd_w, rows, sem):
        wid = lax.axis_index("s") * 2 + lax.axis_index("c")
        lo = wid * NLOC
        zf = jnp.zeros((16,), jnp.float32)
        zi = jnp.zeros((16,), jnp.int32)
        dumpv = jnp.full((16,), NLOC, jnp.int32)
        e0 = jnp.where(lax.iota(jnp.int32, 16) == 0, 1.0, 0.0)

        @pl.loop(0, ACCN, step=128)
        def _(i):
            for u in range(8):
                accn[pl.ds(i + u * 16, 16)] = zf

        @pl.loop(0, ACCD, step=16)
        def _(i):
            accd[pl.ds(i, 16)] = zf

        @pl.loop(0, NBLK)
        def _(b):
            eb = b * SB
            pltpu.sync_copy(src_hbm.at[pl.ds(eb, SB)], st_s)
            pltpu.sync_copy(dst_hbm.at[pl.ds(eb, SB)], st_d)
            pltpu.sync_copy(w_hbm.at[pl.ds(eb, SB)], st_w)

            def scan_body(i, pcount):
                d = st_d[pl.ds(i * 16, 16)]
                m = (d >= lo) & (d < lo + NLOC)
                cnt = plsc.all_reduce_population_count(m)[0]
                s = st_s[pl.ds(i * 16, 16)]
                wv = st_w[pl.ds(i * 16, 16)]
                plsc.store_compressed(pd_s.at[pl.ds(pcount, 16)], s, mask=m)
                plsc.store_compressed(pd_d.at[pl.ds(pcount, 16)], d - lo, mask=m)
                plsc.store_compressed(pd_w.at[pl.ds(pcount, 16)], wv, mask=m)
                return pcount + cnt

            pcount = lax.fori_loop(0, SB // 16, scan_body, jnp.int32(0))

            for u in range(GB // 16):
                pd_s[pl.ds(pcount + u * 16, 16)] = zi
                pd_d[pl.ds(pcount + u * 16, 16)] = dumpv
                pd_w[pl.ds(pcount + u * 16, 16)] = zf
            nb = (pcount + GB - 1) // GB

            def batch_body(bi, _):
                pltpu.async_copy(
                    h_hbm.at[pd_s.at[pl.ds(bi * GB, GB)]], rows, sem
                ).wait()
                for sub in range(GB // 16):
                    dlv = pd_d[pl.ds(bi * GB + sub * 16, 16)]
                    wvv = pd_w[pl.ds(bi * GB + sub * 16, 16)]
                    for j in range(16):
                        r = sub * 16 + j
                        dl = dlv[j]
                        wj = wvv[j]
                        rb = dl * D
                        for c in range(D // 16):
                            plsc.addupdate(
                                accn.at[pl.ds(rb + c * 16, 16)],
                                wj * rows[r, pl.ds(c * 16, 16)],
                            )
                        plsc.addupdate(accd.at[pl.ds(dl * 16, 16)], wj * e0)
                return 0

            lax.fori_loop(0, nb, batch_body, 0)

        pltpu.sync_copy(accn.at[pl.ds(0, NLOC * D)],
                        num_hbm.at[pl.ds(wid * NLOC * D, NLOC * D)])
        pltpu.sync_copy(accd.at[pl.ds(0, NLOC * 16)],
                        den_hbm.at[pl.ds(wid * NLOC * 16, NLOC * 16)])

    return k(h, srcp, dstp, w)


def _tc_layer_in(x, W, Ap):
    R = 1000
    DI, DO = W.shape

    def body(x_ref, w_ref, a_ref, h_ref, al_ref):
        h = lax.dot_general(x_ref[...], w_ref[...], (((1,), (0,)), ((), ())),
                            precision=HI, preferred_element_type=jnp.float32)
        h_ref[...] = h
        al_ref[...] = lax.dot_general(h, a_ref[...], (((1,), (0,)), ((), ())),
                                      precision=HI,
                                      preferred_element_type=jnp.float32)

    return pl.pallas_call(
        body,
        grid=(N // R,),
        in_specs=[
            pl.BlockSpec((R, DI), lambda i: (i, 0)),
            pl.BlockSpec((DI, DO), lambda i: (0, 0)),
            pl.BlockSpec((DO, 128), lambda i: (0, 0)),
        ],
        out_specs=[
            pl.BlockSpec((R, DO), lambda i: (i, 0)),
            pl.BlockSpec((R, 128), lambda i: (i, 0)),
        ],
        out_shape=[
            jax.ShapeDtypeStruct((N, DO), jnp.float32),
            jax.ShapeDtypeStruct((N, 128), jnp.float32),
        ],
    )(x, W, Ap)


def _tc_mid(num, den, b, W, Ap):
    R = 1000
    DI, DO = W.shape

    def body(n_ref, d_ref, b_ref, w_ref, a_ref, h_ref, al_ref):
        den_col = d_ref[...][:, 0:1]
        hin = jnp.maximum(n_ref[...] / (den_col + 1e-16) + b_ref[...], 0.0)
        h = lax.dot_general(hin, w_ref[...], (((1,), (0,)), ((), ())),
                            precision=HI, preferred_element_type=jnp.float32)
        h_ref[...] = h
        al_ref[...] = lax.dot_general(h, a_ref[...], (((1,), (0,)), ((), ())),
                                      precision=HI,
                                      preferred_element_type=jnp.float32)

    return pl.pallas_call(
        body,
        grid=(N // R,),
        in_specs=[
            pl.BlockSpec((R, DI), lambda i: (i, 0)),
            pl.BlockSpec((R, 16), lambda i: (i, 0)),
            pl.BlockSpec((1, DI), lambda i: (0, 0)),
            pl.BlockSpec((DI, DO), lambda i: (0, 0)),
            pl.BlockSpec((DO, 128), lambda i: (0, 0)),
        ],
        out_specs=[
            pl.BlockSpec((R, DO), lambda i: (i, 0)),
            pl.BlockSpec((R, 128), lambda i: (i, 0)),
        ],
        out_shape=[
            jax.ShapeDtypeStruct((N, DO), jnp.float32),
            jax.ShapeDtypeStruct((N, 128), jnp.float32),
        ],
    )(num, den, b, W, Ap)


def _tc_out(num, den, b):
    R = 1000
    DO = num.shape[1]

    def body(n_ref, d_ref, b_ref, o_ref):
        den_col = d_ref[...][:, 0:1]
        o_ref[...] = n_ref[...] / (den_col + 1e-16) + b_ref[...]

    return pl.pallas_call(
        body,
        grid=(N // R,),
        in_specs=[
            pl.BlockSpec((R, DO), lambda i: (i, 0)),
            pl.BlockSpec((R, 16), lambda i: (i, 0)),
            pl.BlockSpec((1, DO), lambda i: (0, 0)),
        ],
        out_specs=pl.BlockSpec((R, DO), lambda i: (i, 0)),
        out_shape=jax.ShapeDtypeStruct((N, DO), jnp.float32),
    )(num, den, b)


def _gat_layer_sc(h, al, srcp, dstp, D):
    al_src_p = jnp.pad(al[:, 0], (0, NPAD - N))
    al_dst_p = jnp.pad(al[:, 1], (0, NPAD - N))
    w = _edge_weights(al_src_p, al_dst_p, srcp, dstp)
    num_f, den_f = _scatter(h, srcp, dstp, w, D)
    num = num_f.reshape(NPAD, D)[:N]
    den = den_f.reshape(NPAD, 16)[:N]
    return num, den


def kernel(x, edge_index, W1, a1_src, a1_dst, b1, W2, a2_src, a2_dst, b2):
    ei = edge_index.astype(jnp.int32)
    loop = jnp.arange(N, dtype=jnp.int32)
    src = jnp.concatenate([ei[0], loop])
    dst = jnp.concatenate([ei[1], loop])
    srcp = jnp.pad(src, (0, E_PAD - E_TOT))
    dstp = jnp.pad(dst, (0, E_PAD - E_TOT), constant_values=N)

    A1p = jnp.zeros((256, 128), jnp.float32)
    A1p = A1p.at[:, 0].set(a1_src).at[:, 1].set(a1_dst)
    A2p = jnp.zeros((128, 128), jnp.float32)
    A2p = A2p.at[:, 0].set(a2_src).at[:, 1].set(a2_dst)

    h1, al1 = _tc_layer_in(x, W1, A1p)
    num1, den1 = _gat_layer_sc(h1, al1, srcp, dstp, 256)
    h2, al2 = _tc_mid(num1, den1, b1.reshape(1, 256), W2, A2p)
    num2, den2 = _gat_layer_sc(h2, al2, srcp, dstp, 128)
    return _tc_out(num2, den2, b2.reshape(1, 128))


# speedup vs baseline: 1.4485x; 1.4485x over previous
import dataclasses
import functools

import jax
import jax.numpy as jnp
from jax import lax
from jax.experimental import pallas as pl
from jax.experimental.pallas import tpu as pltpu
from jax.experimental.pallas import tpu_sc as plsc

N = 10000
NPAD = 10016
NLOC = 313
NW = 32
E_TOT = 330000
E_PAD = 335872
EW = E_PAD // NW
SB = 4096
NBLK = E_PAD // SB
CAP = SB + 16
GB = 32
HI = jax.lax.Precision.HIGHEST


def _mesh():
    return plsc.VectorSubcoreMesh(core_axis_name="c", subcore_axis_name="s")


def _sc_params():
    cp = pltpu.CompilerParams()
    if "needs_layout_passes" in pltpu.CompilerParams.__dataclass_fields__:
        cp = dataclasses.replace(cp, needs_layout_passes=False)
    return cp


def _edge_weights(al_src_p, al_dst_p, srcp, dstp):
    @functools.partial(
        pl.kernel,
        out_type=jax.ShapeDtypeStruct((E_PAD,), jnp.float32),
        mesh=_mesh(),
        compiler_params=_sc_params(),
        scratch_types=[
            pltpu.VMEM((NPAD,), jnp.float32),
            pltpu.VMEM((NPAD,), jnp.float32),
            pltpu.VMEM((EW,), jnp.int32),
            pltpu.VMEM((EW,), jnp.int32),
            pltpu.VMEM((EW,), jnp.float32),
        ],
    )
    def k(als_hbm, ald_hbm, src_hbm, dst_hbm, w_hbm, als_v, ald_v, s_v, d_v, w_v):
        wid = lax.axis_index("s") * 2 + lax.axis_index("c")
        base = wid * EW
        pltpu.sync_copy(als_hbm, als_v)
        pltpu.sync_copy(ald_hbm, ald_v)
        pltpu.sync_copy(src_hbm.at[pl.ds(base, EW)], s_v)
        pltpu.sync_copy(dst_hbm.at[pl.ds(base, EW)], d_v)

        @pl.loop(0, EW, step=16)
        def _(i):
            s = s_v[pl.ds(i, 16)]
            d = d_v[pl.ds(i, 16)]
            t = plsc.load_gather(als_v, [s]) + plsc.load_gather(ald_v, [d])
            t = jnp.where(t >= 0.0, t, 0.2 * t)
            w_v[pl.ds(i, 16)] = jnp.exp(t)

        pltpu.sync_copy(w_v, w_hbm.at[pl.ds(base, EW)])

    return k(al_src_p, al_dst_p, srcp, dstp)


def _scatter(h, srcp, dstp, w, D):
    ACCN = (NLOC + 1) * D
    ACCD = (NLOC + 1) * 16

    @functools.partial(
        pl.kernel,
        out_type=(
            jax.ShapeDtypeStruct((NPAD * D,), jnp.float32),
            jax.ShapeDtypeStruct((NPAD * 16,), jnp.float32),
        ),
        mesh=_mesh(),
        compiler_params=_sc_params(),
        scratch_types=[
            pltpu.VMEM((ACCN,), jnp.float32),
            pltpu.VMEM((ACCD,), jnp.float32),
            pltpu.VMEM((SB,), jnp.int32),
            pltpu.VMEM((SB,), jnp.int32),
            pltpu.VMEM((SB,), jnp.float32),
            pltpu.VMEM((CAP,), jnp.int32),
            pltpu.VMEM((CAP,), jnp.int32),
            pltpu.VMEM((CAP,), jnp.float32),
            pltpu.VMEM((GB, D), jnp.float32),
            pltpu.SemaphoreType.DMA,
        ],
    )
    def k(h_hbm, src_hbm, dst_hbm, w_hbm, num_hbm, den_hbm,
          accn, accd, st_s, st_d, st_w, pd_s, pd_d, pd_w, rows, sem):
        wid = lax.axis_index("s") * 2 + lax.axis_index("c")
        lo = wid * NLOC
        zf = jnp.zeros((16,), jnp.float32)
        zi = jnp.zeros((16,), jnp.int32)
        dumpv = jnp.full((16,), NLOC, jnp.int32)
        e0 = jnp.where(lax.iota(jnp.int32, 16) == 0, 1.0, 0.0)

        @pl.loop(0, ACCN, step=128)
        def _(i):
            for u in range(8):
                accn[pl.ds(i + u * 16, 16)] = zf

        @pl.loop(0, ACCD, step=16)
        def _(i):
            accd[pl.ds(i, 16)] = zf

        @pl.loop(0, NBLK)
        def _(b):
            eb = b * SB
            pltpu.sync_copy(src_hbm.at[pl.ds(eb, SB)], st_s)
            pltpu.sync_copy(dst_hbm.at[pl.ds(eb, SB)], st_d)
            pltpu.sync_copy(w_hbm.at[pl.ds(eb, SB)], st_w)

            def scan_body(i, pcount):
                d = st_d[pl.ds(i * 16, 16)]
                m = (d >= lo) & (d < lo + NLOC)
                cnt = plsc.all_reduce_population_count(m)[0]
                s = st_s[pl.ds(i * 16, 16)]
                wv = st_w[pl.ds(i * 16, 16)]
                plsc.store_compressed(pd_s.at[pl.ds(pcount, 16)], s, mask=m)
                plsc.store_compressed(pd_d.at[pl.ds(pcount, 16)], d - lo, mask=m)
                plsc.store_compressed(pd_w.at[pl.ds(pcount, 16)], wv, mask=m)
                return pcount + cnt

            pcount = lax.fori_loop(0, SB // 16, scan_body, jnp.int32(0))

            for u in range(GB // 16):
                pd_s[pl.ds(pcount + u * 16, 16)] = zi
                pd_d[pl.ds(pcount + u * 16, 16)] = dumpv
                pd_w[pl.ds(pcount + u * 16, 16)] = zf
            nb = (pcount + GB - 1) // GB

            def batch_body(bi, _):
                pltpu.async_copy(
                    h_hbm.at[pd_s.at[pl.ds(bi * GB, GB)]], rows, sem
                ).wait()
                for sub in range(GB // 16):
                    dlv = pd_d[pl.ds(bi * GB + sub * 16, 16)]
                    wvv = pd_w[pl.ds(bi * GB + sub * 16, 16)]
                    for j in range(16):
                        r = sub * 16 + j
                        dl = dlv[j]
                        wj = wvv[j]
                        rb = dl * D
                        for c in range(D // 16):
                            plsc.addupdate(
                                accn.at[pl.ds(rb + c * 16, 16)],
                                wj * rows[r, pl.ds(c * 16, 16)],
                            )
                        plsc.addupdate(accd.at[pl.ds(dl * 16, 16)], wj * e0)
                return 0

            lax.fori_loop(0, nb, batch_body, 0)

        pltpu.sync_copy(accn.at[pl.ds(0, NLOC * D)],
                        num_hbm.at[pl.ds(wid * NLOC * D, NLOC * D)])
        pltpu.sync_copy(accd.at[pl.ds(0, NLOC * 16)],
                        den_hbm.at[pl.ds(wid * NLOC * 16, NLOC * 16)])

    return k(h, srcp, dstp, w)


def _tc_layer_in(x, W, Ap):
    R = 1000
    DI, DO = W.shape

    def body(x_ref, w_ref, a_ref, h_ref, al_ref):
        h = lax.dot_general(x_ref[...], w_ref[...], (((1,), (0,)), ((), ())),
                            precision=HI, preferred_element_type=jnp.float32)
        h_ref[...] = h
        al_ref[...] = lax.dot_general(h, a_ref[...], (((1,), (0,)), ((), ())),
                                      precision=HI,
                                      preferred_element_type=jnp.float32)

    return pl.pallas_call(
        body,
        grid=(N // R,),
        in_specs=[
            pl.BlockSpec((R, DI), lambda i: (i, 0)),
            pl.BlockSpec((DI, DO), lambda i: (0, 0)),
            pl.BlockSpec((DO, 128), lambda i: (0, 0)),
        ],
        out_specs=[
            pl.BlockSpec((R, DO), lambda i: (i, 0)),
            pl.BlockSpec((R, 128), lambda i: (i, 0)),
        ],
        out_shape=[
            jax.ShapeDtypeStruct((N, DO), jnp.float32),
            jax.ShapeDtypeStruct((N, 128), jnp.float32),
        ],
    )(x, W, Ap)


def _tc_mid(num, den, b, W, Ap):
    R = 1000
    DI, DO = W.shape

    def body(n_ref, d_ref, b_ref, w_ref, a_ref, h_ref, al_ref):
        den_col = d_ref[...][:, 0:1]
        hin = jnp.maximum(n_ref[...] / (den_col + 1e-16) + b_ref[...], 0.0)
        h = lax.dot_general(hin, w_ref[...], (((1,), (0,)), ((), ())),
                            precision=HI, preferred_element_type=jnp.float32)
        h_ref[...] = h
        al_ref[...] = lax.dot_general(h, a_ref[...], (((1,), (0,)), ((), ())),
                                      precision=HI,
                                      preferred_element_type=jnp.float32)

    return pl.pallas_call(
        body,
        grid=(N // R,),
        in_specs=[
            pl.BlockSpec((R, DI), lambda i: (i, 0)),
            pl.BlockSpec((R, 16), lambda i: (i, 0)),
            pl.BlockSpec((1, DI), lambda i: (0, 0)),
            pl.BlockSpec((DI, DO), lambda i: (0, 0)),
            pl.BlockSpec((DO, 128), lambda i: (0, 0)),
        ],
        out_specs=[
            pl.BlockSpec((R, DO), lambda i: (i, 0)),
            pl.BlockSpec((R, 128), lambda i: (i, 0)),
        ],
        out_shape=[
            jax.ShapeDtypeStruct((N, DO), jnp.float32),
            jax.ShapeDtypeStruct((N, 128), jnp.float32),
        ],
    )(num, den, b, W, Ap)


def _tc_out(num, den, b):
    R = 1000
    DO = num.shape[1]

    def body(n_ref, d_ref, b_ref, o_ref):
        den_col = d_ref[...][:, 0:1]
        o_ref[...] = n_ref[...] / (den_col + 1e-16) + b_ref[...]

    return pl.pallas_call(
        body,
        grid=(N // R,),
        in_specs=[
            pl.BlockSpec((R, DO), lambda i: (i, 0)),
            pl.BlockSpec((R, 16), lambda i: (i, 0)),
            pl.BlockSpec((1, DO), lambda i: (0, 0)),
        ],
        out_specs=pl.BlockSpec((R, DO), lambda i: (i, 0)),
        out_shape=jax.ShapeDtypeStruct((N, DO), jnp.float32),
    )(num, den, b)


def _gat_layer_sc(h, al, srcp, dstp, D):
    al_src_p = jnp.pad(al[:, 0], (0, NPAD - N))
    al_dst_p = jnp.pad(al[:, 1], (0, NPAD - N))
    w = _edge_weights(al_src_p, al_dst_p, srcp, dstp)
    num_f, den_f = _scatter(h, srcp, dstp, w, D)
    num = num_f.reshape(NPAD, D)[:N]
    den = den_f.reshape(NPAD, 16)[:N]
    return num, den


def kernel(x, edge_index, W1, a1_src, a1_dst, b1, W2, a2_src, a2_dst, b2):
    ei = edge_index.astype(jnp.int32)
    loop = jnp.arange(N, dtype=jnp.int32)
    src = jnp.concatenate([ei[0], loop])
    dst = jnp.concatenate([ei[1], loop])
    srcp = jnp.pad(src, (0, E_PAD - E_TOT))
    dstp = jnp.pad(dst, (0, E_PAD - E_TOT), constant_values=N)

    A1p = jnp.zeros((256, 128), jnp.float32)
    A1p = A1p.at[:, 0].set(a1_src).at[:, 1].set(a1_dst)
    A2p = jnp.zeros((128, 128), jnp.float32)
    A2p = A2p.at[:, 0].set(a2_src).at[:, 1].set(a2_dst)

    h1, al1 = _tc_layer_in(x, W1, A1p)
    num1, den1 = _gat_layer_sc(h1, al1, srcp, dstp, 256)
    h2, al2 = _tc_mid(num1, den1, b1.reshape(1, 256), W2, A2p)
    num2, den2 = _gat_layer_sc(h2, al2, srcp, dstp, 128)
    return _tc_out(num2, den2, b2.reshape(1, 128))


# speedup vs baseline: 1.4784x; 1.0207x over previous
import dataclasses
import functools

import jax
import jax.numpy as jnp
from jax import lax
from jax.experimental import pallas as pl
from jax.experimental.pallas import tpu as pltpu
from jax.experimental.pallas import tpu_sc as plsc

N = 10000
NPAD = 10016
NLOC = 313
NW = 32
E_TOT = 330000
E_PAD = 335872
EW = E_PAD // NW
SB = 4096
NBLK = E_PAD // SB
CAP = SB + 16
GB = 32
HI = jax.lax.Precision.HIGHEST


def _mesh():
    return plsc.VectorSubcoreMesh(core_axis_name="c", subcore_axis_name="s")


def _sc_params():
    cp = pltpu.CompilerParams()
    if "needs_layout_passes" in pltpu.CompilerParams.__dataclass_fields__:
        cp = dataclasses.replace(cp, needs_layout_passes=False)
    return cp


def _edge_weights(al_src_p, al_dst_p, srcp, dstp):
    @functools.partial(
        pl.kernel,
        out_type=jax.ShapeDtypeStruct((E_PAD,), jnp.float32),
        mesh=_mesh(),
        compiler_params=_sc_params(),
        scratch_types=[
            pltpu.VMEM((NPAD,), jnp.float32),
            pltpu.VMEM((NPAD,), jnp.float32),
            pltpu.VMEM((EW,), jnp.int32),
            pltpu.VMEM((EW,), jnp.int32),
            pltpu.VMEM((EW,), jnp.float32),
        ],
    )
    def k(als_hbm, ald_hbm, src_hbm, dst_hbm, w_hbm, als_v, ald_v, s_v, d_v, w_v):
        wid = lax.axis_index("s") * 2 + lax.axis_index("c")
        base = wid * EW
        pltpu.sync_copy(als_hbm, als_v)
        pltpu.sync_copy(ald_hbm, ald_v)
        pltpu.sync_copy(src_hbm.at[pl.ds(base, EW)], s_v)
        pltpu.sync_copy(dst_hbm.at[pl.ds(base, EW)], d_v)

        @pl.loop(0, EW, step=16)
        def _(i):
            s = s_v[pl.ds(i, 16)]
            d = d_v[pl.ds(i, 16)]
            t = plsc.load_gather(als_v, [s]) + plsc.load_gather(ald_v, [d])
            t = jnp.where(t >= 0.0, t, 0.2 * t)
            w_v[pl.ds(i, 16)] = jnp.exp(t)

        pltpu.sync_copy(w_v, w_hbm.at[pl.ds(base, EW)])

    return k(al_src_p, al_dst_p, srcp, dstp)


def _scatter(h, srcp, dstp, w, D):
    ACCN = (NLOC + 1) * D
    ACCD = (NLOC + 1) * 16

    @functools.partial(
        pl.kernel,
        out_type=(
            jax.ShapeDtypeStruct((NPAD * D,), jnp.float32),
            jax.ShapeDtypeStruct((NPAD * 16,), jnp.float32),
        ),
        mesh=_mesh(),
        compiler_params=_sc_params(),
        scratch_types=[
            pltpu.VMEM((ACCN,), jnp.float32),
            pltpu.VMEM((ACCD,), jnp.float32),
            pltpu.VMEM((SB,), jnp.int32),
            pltpu.VMEM((SB,), jnp.int32),
            pltpu.VMEM((SB,), jnp.float32),
            pltpu.VMEM((CAP,), jnp.int32),
            pltpu.VMEM((CAP,), jnp.int32),
            pltpu.VMEM((CAP,), jnp.float32),
            pltpu.VMEM((GB, D), jnp.float32),
            pltpu.SemaphoreType.DMA,
        ],
    )
    def k(h_hbm, src_hbm, dst_hbm, w_hbm, num_hbm, den_hbm,
          accn, accd, st_s, st_d, st_w, pd_s, pd_d, pd_w, rows, sem):
        wid = lax.axis_index("s") * 2 + lax.axis_index("c")
        lo = wid * NLOC
        zf = jnp.zeros((16,), jnp.float32)
        zi = jnp.zeros((16,), jnp.int32)
        dumpv = jnp.full((16,), NLOC * D, jnp.int32)
        e0 = jnp.where(lax.iota(jnp.int32, 16) == 0, 1.0, 0.0)

        @pl.loop(0, ACCN, step=128)
        def _(i):
            for u in range(8):
                accn[pl.ds(i + u * 16, 16)] = zf

        @pl.loop(0, ACCD, step=16)
        def _(i):
            accd[pl.ds(i, 16)] = zf

        @pl.loop(0, NBLK)
        def _(b):
            eb = b * SB
            pltpu.sync_copy(src_hbm.at[pl.ds(eb, SB)], st_s)
            pltpu.sync_copy(dst_hbm.at[pl.ds(eb, SB)], st_d)
            pltpu.sync_copy(w_hbm.at[pl.ds(eb, SB)], st_w)

            def scan_body(i, pcount):
                d = st_d[pl.ds(i * 16, 16)]
                m = (d >= lo) & (d < lo + NLOC)
                cnt = plsc.all_reduce_population_count(m)[0]
                s = st_s[pl.ds(i * 16, 16)]
                wv = st_w[pl.ds(i * 16, 16)]
                plsc.store_compressed(pd_s.at[pl.ds(pcount, 16)], s, mask=m)
                plsc.store_compressed(pd_d.at[pl.ds(pcount, 16)], d - lo, mask=m)
                plsc.store_compressed(pd_w.at[pl.ds(pcount, 16)], wv, mask=m)
                return pcount + cnt

            pcount = lax.fori_loop(0, SB // 16, scan_body, jnp.int32(0))

            for u in range(GB // 16):
                pd_s[pl.ds(pcount + u * 16, 16)] = zi
                pd_d[pl.ds(pcount + u * 16, 16)] = dumpv
                pd_w[pl.ds(pcount + u * 16, 16)] = zf
            nb = (pcount + GB - 1) // GB

            def batch_body(bi, _):
                pltpu.async_copy(
                    h_hbm.at[pd_s.at[pl.ds(bi * GB, GB)]], rows, sem
                ).wait()
                for sub in range(GB // 16):
                    dlv = pd_d[pl.ds(bi * GB + sub * 16, 16)]
                    wvv = pd_w[pl.ds(bi * GB + sub * 16, 16)]
                    sh = (D // 16).bit_length() - 1
                    for j in range(16):
                        r = sub * 16 + j
                        rb = dlv[j]
                        wj = wvv[j]
                        for c in range(D // 16):
                            plsc.addupdate(
                                accn.at[pl.ds(rb + c * 16, 16)],
                                wj * rows[r, pl.ds(c * 16, 16)],
                            )
                        plsc.addupdate(accd.at[pl.ds(rb >> sh, 16)], wj * e0)
                return 0

            lax.fori_loop(0, nb, batch_body, 0)

        pltpu.sync_copy(accn.at[pl.ds(0, NLOC * D)],
                        num_hbm.at[pl.ds(wid * NLOC * D, NLOC * D)])
        pltpu.sync_copy(accd.at[pl.ds(0, NLOC * 16)],
                        den_hbm.at[pl.ds(wid * NLOC * 16, NLOC * 16)])

    return k(h, srcp, dstp, w)


def _tc_layer_in(x, W, Ap):
    R = 1000
    DI, DO = W.shape

    def body(x_ref, w_ref, a_ref, h_ref, al_ref):
        h = lax.dot_general(x_ref[...], w_ref[...], (((1,), (0,)), ((), ())),
                            precision=HI, preferred_element_type=jnp.float32)
        h_ref[...] = h
        al_ref[...] = lax.dot_general(h, a_ref[...], (((1,), (0,)), ((), ())),
                                      precision=HI,
                                      preferred_element_type=jnp.float32)

    return pl.pallas_call(
        body,
        grid=(N // R,),
        in_specs=[
            pl.BlockSpec((R, DI), lambda i: (i, 0)),
            pl.BlockSpec((DI, DO), lambda i: (0, 0)),
            pl.BlockSpec((DO, 128), lambda i: (0, 0)),
        ],
        out_specs=[
            pl.BlockSpec((R, DO), lambda i: (i, 0)),
            pl.BlockSpec((R, 128), lambda i: (i, 0)),
        ],
        out_shape=[
            jax.ShapeDtypeStruct((N, DO), jnp.float32),
            jax.ShapeDtypeStruct((N, 128), jnp.float32),
        ],
    )(x, W, Ap)


def _tc_mid(num, den, b, W, Ap):
    R = 1000
    DI, DO = W.shape

    def body(n_ref, d_ref, b_ref, w_ref, a_ref, h_ref, al_ref):
        den_col = d_ref[...][:, 0:1]
        hin = jnp.maximum(n_ref[...] / (den_col + 1e-16) + b_ref[...], 0.0)
        h = lax.dot_general(hin, w_ref[...], (((1,), (0,)), ((), ())),
                            precision=HI, preferred_element_type=jnp.float32)
        h_ref[...] = h
        al_ref[...] = lax.dot_general(h, a_ref[...], (((1,), (0,)), ((), ())),
                                      precision=HI,
                                      preferred_element_type=jnp.float32)

    return pl.pallas_call(
        body,
        grid=(N // R,),
        in_specs=[
            pl.BlockSpec((R, DI), lambda i: (i, 0)),
            pl.BlockSpec((R, 16), lambda i: (i, 0)),
            pl.BlockSpec((1, DI), lambda i: (0, 0)),
            pl.BlockSpec((DI, DO), lambda i: (0, 0)),
            pl.BlockSpec((DO, 128), lambda i: (0, 0)),
        ],
        out_specs=[
            pl.BlockSpec((R, DO), lambda i: (i, 0)),
            pl.BlockSpec((R, 128), lambda i: (i, 0)),
        ],
        out_shape=[
            jax.ShapeDtypeStruct((N, DO), jnp.float32),
            jax.ShapeDtypeStruct((N, 128), jnp.float32),
        ],
    )(num, den, b, W, Ap)


def _tc_out(num, den, b):
    R = 1000
    DO = num.shape[1]

    def body(n_ref, d_ref, b_ref, o_ref):
        den_col = d_ref[...][:, 0:1]
        o_ref[...] = n_ref[...] / (den_col + 1e-16) + b_ref[...]

    return pl.pallas_call(
        body,
        grid=(N // R,),
        in_specs=[
            pl.BlockSpec((R, DO), lambda i: (i, 0)),
            pl.BlockSpec((R, 16), lambda i: (i, 0)),
            pl.BlockSpec((1, DO), lambda i: (0, 0)),
        ],
        out_specs=pl.BlockSpec((R, DO), lambda i: (i, 0)),
        out_shape=jax.ShapeDtypeStruct((N, DO), jnp.float32),
    )(num, den, b)


def _gat_layer_sc(h, al, srcp, dstp, D):
    al_src_p = jnp.pad(al[:, 0], (0, NPAD - N))
    al_dst_p = jnp.pad(al[:, 1], (0, NPAD - N))
    w = _edge_weights(al_src_p, al_dst_p, srcp, dstp)
    num_f, den_f = _scatter(h, srcp, dstp, w, D)
    num = num_f.reshape(NPAD, D)[:N]
    den = den_f.reshape(NPAD, 16)[:N]
    return num, den


def kernel(x, edge_index, W1, a1_src, a1_dst, b1, W2, a2_src, a2_dst, b2):
    ei = edge_index.astype(jnp.int32)
    loop = jnp.arange(N, dtype=jnp.int32)
    src = jnp.concatenate([ei[0], loop])
    dst = jnp.concatenate([ei[1], loop])
    srcp = jnp.pad(src, (0, E_PAD - E_TOT))
    dstp = jnp.pad(dst, (0, E_PAD - E_TOT), constant_values=N)

    A1p = jnp.zeros((256, 128), jnp.float32)
    A1p = A1p.at[:, 0].set(a1_src).at[:, 1].set(a1_dst)
    A2p = jnp.zeros((128, 128), jnp.float32)
    A2p = A2p.at[:, 0].set(a2_src).at[:, 1].set(a2_dst)

    h1, al1 = _tc_layer_in(x, W1, A1p)
    num1, den1 = _gat_layer_sc(h1, al1, srcp, dstp, 256)
    h2, al2 = _tc_mid(num1, den1, b1.reshape(1, 256), W2, A2p)
    num2, den2 = _gat_layer_sc(h2, al2, srcp, dstp, 128)
    return _tc_out(num2, den2, b2.reshape(1, 128))
